# Initial kernel scaffold; baseline (speedup 1.0000x reference)
import jax
import jax.numpy as jnp
from jax.experimental import pallas as pl


def kernel(image_features, text_embedding, W, b):
    raise NotImplementedError("write your pallas kernel here")


# speedup vs baseline: 1.1655x; 1.1655x over previous
import functools

import jax
import jax.numpy as jnp
from jax import lax
from jax.experimental import pallas as pl
from jax.experimental.pallas import tpu as pltpu
from jax.experimental.pallas import tpu_sc as plsc

_B, _N, _D, _TD = 64, 576, 768, 512
_K = 144
_DEBUG_JAX_GATHER = False


def _score_body(x_ref, text_ref, w_ref, bias_ref, s_ref, inv_ref):
    x = x_ref[0]
    t = lax.dot_general(text_ref[0], w_ref[...], (((1,), (1,)), ((), ())),
                        preferred_element_type=jnp.float32,
                        precision=lax.Precision.DEFAULT)
    t = t + bias_ref[...]
    tn = t / jnp.sqrt(jnp.sum(t * t))
    xx = x * x
    ones = jnp.ones((_D, 1), dtype=jnp.float32)
    norm2 = lax.dot_general(xx, ones, (((1,), (0,)), ((), ())),
                            preferred_element_type=jnp.float32,
                            precision=lax.Precision.HIGHEST)
    inv = 1.0 / jnp.sqrt(norm2)
    img = x * inv
    imgb = img.astype(jnp.bfloat16).astype(jnp.float32)
    tnb = tn.astype(jnp.bfloat16).astype(jnp.float32)
    s = lax.dot_general(imgb, tnb, (((1,), (1,)), ((), ())),
                        preferred_element_type=jnp.float32,
                        precision=lax.Precision.HIGHEST)
    s_ref[0] = s
    inv_ref[0] = inv


def _stage_a(image_features, text_embedding, w, bias2d):
    s3, inv3 = pl.pallas_call(
        _score_body,
        grid=(_B,),
        in_specs=[
            pl.BlockSpec((1, _N, _D), lambda b: (b, 0, 0)),
            pl.BlockSpec((1, 1, _TD), lambda b: (b, 0, 0)),
            pl.BlockSpec((_D, _TD), lambda b: (0, 0)),
            pl.BlockSpec((1, _D), lambda b: (0, 0)),
        ],
        out_specs=[
            pl.BlockSpec((1, _N, 1), lambda b: (b, 0, 0)),
            pl.BlockSpec((1, _N, 1), lambda b: (b, 0, 0)),
        ],
        out_shape=[
            jax.ShapeDtypeStruct((_B, _N, 1), jnp.float32),
            jax.ShapeDtypeStruct((_B, _N, 1), jnp.float32),
        ],
    )(image_features, text_embedding.reshape(_B, 1, _TD), w, bias2d)
    return s3.reshape(_B, _N), inv3.reshape(_B, _N)


def _topk_body(s_ref, mask_ref):
    s = s_ref[...]
    mx = jnp.max(s, axis=-1, keepdims=True)
    e = jnp.exp(s - mx)
    p = e / jnp.sum(e, axis=-1, keepdims=True)
    ukey = lax.bitcast_convert_type(p, jnp.int32)
    t = jnp.zeros((_B, 1), jnp.int32)
    for bit in range(30, -1, -1):
        cand = t | (1 << bit)
        cnt = jnp.sum((ukey >= cand).astype(jnp.int32), axis=1, keepdims=True)
        t = jnp.where(cnt >= _K, cand, t)
    gt = ukey > t
    tie = ukey == t
    c_gt = jnp.sum(gt.astype(jnp.int32), axis=1, keepdims=True)
    r_need = _K - c_gt
    idx = lax.broadcasted_iota(jnp.int32, (_B, _N), 1)
    key2 = jnp.where(tie, 1023 - idx, -1)
    t2 = jnp.zeros((_B, 1), jnp.int32)
    for bit in range(9, -1, -1):
        cand = t2 | (1 << bit)
        cnt = jnp.sum((key2 >= cand).astype(jnp.int32), axis=1, keepdims=True)
        t2 = jnp.where(cnt >= r_need, cand, t2)
    istar = 1023 - t2
    mask = gt | (tie & (idx <= istar))
    mask_ref[...] = mask.astype(jnp.int32)


def _stage_b(scores):
    return pl.pallas_call(
        _topk_body,
        out_shape=jax.ShapeDtypeStruct((_B, _N), jnp.int32),
    )(scores)


_NC, _NS, _L = 2, 16, 16
_ROWS_PER_W = _B // (_NC * _NS)
_HALF = _K // 2


def _sc_body(x_hbm, mask_hbm, inv_hbm, out_hbm,
             mask_v, inv_v, idx_v, invsel_v, rows_v, sem):
    wid = lax.axis_index("s") * _NC + lax.axis_index("c")
    for i in range(_ROWS_PER_W):
        row = wid * _ROWS_PER_W + i
        pltpu.sync_copy(mask_hbm.at[row], mask_v)
        pltpu.sync_copy(inv_hbm.at[row], inv_v)

        def _chunk(c, off):
            mi = mask_v[pl.ds(c * _L, _L)]
            m = mi != 0
            pre = plsc.cumsum(mi)
            pos = off + pre - 1
            gidx = lax.iota(jnp.int32, _L) + (c * _L + row * _N)
            plsc.store_scatter(idx_v, [pos], gidx, mask=m)
            return off + jnp.sum(mi)

        lax.fori_loop(0, _N // _L, _chunk, jnp.int32(0))

        def _inv_chunk(c, _):
            g = idx_v[pl.ds(c * _L, _L)]
            loc = g - row * _N
            invsel_v[pl.ds(c * _L, _L)] = plsc.load_gather(inv_v, [loc])
            return 0

        lax.fori_loop(0, _K // _L, _inv_chunk, 0)

        cp0 = pltpu.async_copy(x_hbm.at[idx_v.at[pl.ds(0, _HALF)]],
                               rows_v.at[pl.ds(0, _HALF)], sem)
        cp1 = pltpu.async_copy(x_hbm.at[idx_v.at[pl.ds(_HALF, _HALF)]],
                               rows_v.at[pl.ds(_HALF, _HALF)], sem)
        cp0.wait()
        cp1.wait()

        def _srow(r, _):
            s = invsel_v[pl.ds(r, _L)][0]
            for c in range(_D // _L):
                sl = pl.ds(c * _L, _L)
                rows_v[r, sl] = rows_v[r, sl] * s
            return 0

        lax.fori_loop(0, _K, _srow, 0)

        pltpu.sync_copy(rows_v, out_hbm.at[pl.ds(row * _K, _K)])


def _stage_c(x_flat, mask_i, inv):
    mesh = plsc.VectorSubcoreMesh(core_axis_name="c", subcore_axis_name="s")
    f = functools.partial(
        pl.kernel,
        mesh=mesh,
        compiler_params=pltpu.CompilerParams(needs_layout_passes=False),
        out_type=jax.ShapeDtypeStruct((_B * _K, _D), jnp.float32),
        scratch_types=[
            pltpu.VMEM((_N,), jnp.int32),
            pltpu.VMEM((_N,), jnp.float32),
            pltpu.VMEM((_K,), jnp.int32),
            pltpu.VMEM((_K + _L,), jnp.float32),
            pltpu.VMEM((_K, _D), jnp.float32),
            pltpu.SemaphoreType.DMA,
        ],
    )(_sc_body)
    return f(x_flat, mask_i, inv)


_DEBUG_JAX_SCORES = False


def kernel(image_features, text_embedding, W, b):
    if _DEBUG_JAX_SCORES:
        proj = text_embedding @ W.T + b
        img_n = jnp.linalg.norm(image_features, axis=-1, keepdims=True)
        txt = proj / jnp.linalg.norm(proj, axis=-1, keepdims=True)
        img = image_features / img_n
        scores = jnp.squeeze(img @ jnp.swapaxes(txt[:, None, :], -2, -1), -1)
        inv = (1.0 / img_n)[..., 0]
    else:
        scores, inv = _stage_a(image_features, text_embedding, W,
                               b.reshape(1, _D))
    mask_i = _stage_b(scores)
    if _DEBUG_JAX_GATHER:
        idx = jnp.argsort(-mask_i, axis=1, stable=True)[:, :_K]
        sidx = jnp.sort(idx, axis=1)
        sel = jnp.take_along_axis(image_features, sidx[..., None], axis=1)
        sel = sel * jnp.take_along_axis(inv, sidx, axis=1)[..., None]
        return sel, mask_i.astype(bool)
    x_flat = image_features.reshape(_B * _N, _D)
    out = _stage_c(x_flat, mask_i, inv)
    return out.reshape(_B, _K, _D), mask_i.astype(bool)


# speedup vs baseline: 1.6141x; 1.3849x over previous
import functools

import jax
import jax.numpy as jnp
from jax import lax
from jax.experimental import pallas as pl
from jax.experimental.pallas import tpu as pltpu
from jax.experimental.pallas import tpu_sc as plsc

_B, _N, _D, _TD = 64, 576, 768, 512
_K = 144
_DEBUG_JAX_GATHER = False


def _score_body(x_ref, text_ref, w_ref, bias_ref, s_ref, inv_ref):
    x = x_ref[0]
    t = lax.dot_general(text_ref[0], w_ref[...], (((1,), (1,)), ((), ())),
                        preferred_element_type=jnp.float32,
                        precision=lax.Precision.DEFAULT)
    t = t + bias_ref[...]
    tn = t / jnp.sqrt(jnp.sum(t * t))
    norm2 = jnp.sum(x * x, axis=1, keepdims=True)
    inv = 1.0 / jnp.sqrt(norm2)
    img = x * inv
    imgb = img.astype(jnp.bfloat16).astype(jnp.float32)
    tnb = tn.astype(jnp.bfloat16).astype(jnp.float32)
    s = lax.dot_general(imgb, tnb, (((1,), (1,)), ((), ())),
                        preferred_element_type=jnp.float32,
                        precision=lax.Precision.HIGHEST)
    s_ref[0] = s
    inv_ref[0] = inv


def _stage_a(image_features, text_embedding, w, bias2d):
    s3, inv3 = pl.pallas_call(
        _score_body,
        grid=(_B,),
        in_specs=[
            pl.BlockSpec((1, _N, _D), lambda b: (b, 0, 0)),
            pl.BlockSpec((1, 1, _TD), lambda b: (b, 0, 0)),
            pl.BlockSpec((_D, _TD), lambda b: (0, 0)),
            pl.BlockSpec((1, _D), lambda b: (0, 0)),
        ],
        out_specs=[
            pl.BlockSpec((1, _N, 1), lambda b: (b, 0, 0)),
            pl.BlockSpec((1, _N, 1), lambda b: (b, 0, 0)),
        ],
        out_shape=[
            jax.ShapeDtypeStruct((_B, _N, 1), jnp.float32),
            jax.ShapeDtypeStruct((_B, _N, 1), jnp.float32),
        ],
    )(image_features, text_embedding.reshape(_B, 1, _TD), w, bias2d)
    return s3.reshape(_B, _N), inv3.reshape(_B, _N)


def _topk_body(s_ref, mask_ref):
    s = s_ref[...]
    mx = jnp.max(s, axis=-1, keepdims=True)
    e = jnp.exp(s - mx)
    p = e / jnp.sum(e, axis=-1, keepdims=True)
    ukey = lax.bitcast_convert_type(p, jnp.int32)
    t = jnp.zeros((_B, 1), jnp.int32)
    for bit in range(30, -1, -1):
        cand = t | (1 << bit)
        cnt = jnp.sum((ukey >= cand).astype(jnp.int32), axis=1, keepdims=True)
        t = jnp.where(cnt >= _K, cand, t)
    gt = ukey > t
    tie = ukey == t
    c_gt = jnp.sum(gt.astype(jnp.int32), axis=1, keepdims=True)
    r_need = _K - c_gt
    idx = lax.broadcasted_iota(jnp.int32, (_B, _N), 1)
    key2 = jnp.where(tie, 1023 - idx, -1)
    t2 = jnp.zeros((_B, 1), jnp.int32)
    for bit in range(9, -1, -1):
        cand = t2 | (1 << bit)
        cnt = jnp.sum((key2 >= cand).astype(jnp.int32), axis=1, keepdims=True)
        t2 = jnp.where(cnt >= r_need, cand, t2)
    istar = 1023 - t2
    mask = gt | (tie & (idx <= istar))
    mask_ref[...] = mask.astype(jnp.int32)


def _stage_b(scores):
    return pl.pallas_call(
        _topk_body,
        out_shape=jax.ShapeDtypeStruct((_B, _N), jnp.int32),
    )(scores)


_NC, _NS, _L = 2, 16, 16
_ROWS_PER_W = _B // (_NC * _NS)
_HALF = _K // 2


def _sc_body(x_hbm, mask_hbm, inv_hbm, out_hbm,
             mask_v, inv_v, idx_v, invsel_v, rows_v, sem):
    wid = lax.axis_index("s") * _NC + lax.axis_index("c")
    for i in range(_ROWS_PER_W):
        row = wid * _ROWS_PER_W + i
        pltpu.sync_copy(mask_hbm.at[row], mask_v)
        pltpu.sync_copy(inv_hbm.at[row], inv_v)

        def _chunk(c, off):
            mi = mask_v[pl.ds(c * _L, _L)]
            m = mi != 0
            pre = plsc.cumsum(mi)
            pos = off + pre - 1
            gidx = lax.iota(jnp.int32, _L) + (c * _L + row * _N)
            plsc.store_scatter(idx_v, [pos], gidx, mask=m)
            return off + jnp.sum(mi)

        lax.fori_loop(0, _N // _L, _chunk, jnp.int32(0))

        def _inv_chunk(c, _):
            g = idx_v[pl.ds(c * _L, _L)]
            loc = g - row * _N
            invsel_v[pl.ds(c * _L, _L)] = plsc.load_gather(inv_v, [loc])
            return 0

        lax.fori_loop(0, _K // _L, _inv_chunk, 0)

        cp0 = pltpu.async_copy(x_hbm.at[idx_v.at[pl.ds(0, _HALF)]],
                               rows_v.at[pl.ds(0, _HALF)], sem)
        cp1 = pltpu.async_copy(x_hbm.at[idx_v.at[pl.ds(_HALF, _HALF)]],
                               rows_v.at[pl.ds(_HALF, _HALF)], sem)
        cp0.wait()
        cp1.wait()

        def _srow(r, _):
            s = invsel_v[pl.ds(r, _L)][0]
            for c in range(_D // _L):
                sl = pl.ds(c * _L, _L)
                rows_v[r, sl] = rows_v[r, sl] * s
            return 0

        lax.fori_loop(0, _K, _srow, 0)

        pltpu.sync_copy(rows_v, out_hbm.at[pl.ds(row * _K, _K)])


def _stage_c(x_flat, mask_i, inv):
    mesh = plsc.VectorSubcoreMesh(core_axis_name="c", subcore_axis_name="s")
    f = functools.partial(
        pl.kernel,
        mesh=mesh,
        compiler_params=pltpu.CompilerParams(needs_layout_passes=False),
        out_type=jax.ShapeDtypeStruct((_B * _K, _D), jnp.float32),
        scratch_types=[
            pltpu.VMEM((_N,), jnp.int32),
            pltpu.VMEM((_N,), jnp.float32),
            pltpu.VMEM((_K,), jnp.int32),
            pltpu.VMEM((_K + _L,), jnp.float32),
            pltpu.VMEM((_K, _D), jnp.float32),
            pltpu.SemaphoreType.DMA,
        ],
    )(_sc_body)
    return f(x_flat, mask_i, inv)


_DEBUG_JAX_SCORES = False


def kernel(image_features, text_embedding, W, b):
    if _DEBUG_JAX_SCORES:
        proj = text_embedding @ W.T + b
        img_n = jnp.linalg.norm(image_features, axis=-1, keepdims=True)
        txt = proj / jnp.linalg.norm(proj, axis=-1, keepdims=True)
        img = image_features / img_n
        scores = jnp.squeeze(img @ jnp.swapaxes(txt[:, None, :], -2, -1), -1)
        inv = (1.0 / img_n)[..., 0]
    else:
        scores, inv = _stage_a(image_features, text_embedding, W,
                               b.reshape(1, _D))
    mask_i = _stage_b(scores)
    if _DEBUG_JAX_GATHER:
        idx = jnp.argsort(-mask_i, axis=1, stable=True)[:, :_K]
        sidx = jnp.sort(idx, axis=1)
        sel = jnp.take_along_axis(image_features, sidx[..., None], axis=1)
        sel = sel * jnp.take_along_axis(inv, sidx, axis=1)[..., None]
        return sel, mask_i.astype(bool)
    x_flat = image_features.reshape(_B * _N, _D)
    out = _stage_c(x_flat, mask_i, inv)
    return out.reshape(_B, _K, _D), mask_i.astype(bool)
